# Initial kernel scaffold; baseline (speedup 1.0000x reference)
import jax
import jax.numpy as jnp
from jax.experimental import pallas as pl


def kernel(x, edge_index, batch, W1, b1, W2, b2, W3, b3, fW1, fb1, fW2, fb2):
    raise NotImplementedError("write your pallas kernel here")


# speedup vs baseline: 6.1465x; 6.1465x over previous
import functools

import jax
import jax.numpy as jnp
from jax import lax
from jax.experimental import pallas as pl
from jax.experimental.pallas import tpu as pltpu
from jax.experimental.pallas import tpu_sc as plsc

_N = 10000
_D = 128
_G = 16
_E = 320000
_NC = 2
_NS = 16
_CH = 128
_EPT = 10240
_EP = _NC * _NS * _EPT
_NCHUNK = _EPT // _CH
_NR = 10240
_PAD_DST = 10008
_RPT = _NR // _NS
_DW = 16
_RB = 1000


def _sc_mesh():
    return plsc.VectorSubcoreMesh(core_axis_name="c", subcore_axis_name="s")


def _sc_deg_body(dst_hbm, ones_hbm, deg_out, deg_sh, ones_v, idx_v):
    c = lax.axis_index("c")
    s = lax.axis_index("s")
    pltpu.sync_copy(ones_hbm.at[pl.ds(0, _CH)], ones_v)
    pltpu.sync_copy(ones_hbm, deg_sh.at[pl.ds(s * _RPT, _RPT)])
    plsc.subcore_barrier()
    tbase = c * (_EP // _NC) + s * _EPT

    def body(g, carry):
        base = tbase + g * _CH
        pltpu.sync_copy(dst_hbm.at[pl.ds(base, _CH)], idx_v.at[0])
        pltpu.sync_copy(ones_v, deg_sh.at[idx_v.at[0]], add=True)
        return carry

    lax.fori_loop(0, _NCHUNK, body, 0)
    plsc.subcore_barrier()
    pltpu.sync_copy(deg_sh.at[pl.ds(s * _RPT, _RPT)],
                    deg_out.at[c, pl.ds(s * _RPT, _RPT)])


def _make_sc_deg():
    return pl.kernel(
        _sc_deg_body,
        out_type=jax.ShapeDtypeStruct((_NC, _NR, _DW), jnp.float32),
        mesh=_sc_mesh(),
        scratch_types=[
            pltpu.VMEM_SHARED((_NR, _DW), jnp.float32),
            pltpu.VMEM((_CH, _DW), jnp.float32),
            pltpu.VMEM((1, _CH), jnp.int32),
        ],
        name="sc_deg",
    )


def _sc_agg_body(u_hbm, src_hbm, dst_hbm, acc_out, acc_sh, sidx_v, didx_v, rows_v):
    c = lax.axis_index("c")
    s = lax.axis_index("s")
    pltpu.sync_copy(u_hbm.at[pl.ds(s * _RPT, _RPT)], acc_sh.at[pl.ds(s * _RPT, _RPT)])
    plsc.subcore_barrier()
    tbase = c * (_EP // _NC) + s * _EPT

    def body(g, carry):
        base = tbase + g * _CH
        pltpu.sync_copy(src_hbm.at[pl.ds(base, _CH)], sidx_v.at[0])
        pltpu.sync_copy(dst_hbm.at[pl.ds(base, _CH)], didx_v.at[0])
        pltpu.sync_copy(u_hbm.at[sidx_v.at[0]], rows_v)
        pltpu.sync_copy(rows_v, acc_sh.at[didx_v.at[0]], add=True)
        return carry

    lax.fori_loop(0, _NCHUNK, body, 0)
    plsc.subcore_barrier()
    pltpu.sync_copy(acc_sh.at[pl.ds(s * _RPT, _RPT)],
                    acc_out.at[c, pl.ds(s * _RPT, _RPT)])


def _make_sc_agg():
    return pl.kernel(
        _sc_agg_body,
        out_type=jax.ShapeDtypeStruct((_NC, _NR, _D), jnp.float32),
        mesh=_sc_mesh(),
        scratch_types=[
            pltpu.VMEM_SHARED((_NR, _D), jnp.float32),
            pltpu.VMEM((1, _CH), jnp.int32),
            pltpu.VMEM((1, _CH), jnp.int32),
            pltpu.VMEM((_CH, _D), jnp.float32),
        ],
        name="sc_agg",
    )


def _dinv_block(deg_ref):
    d = deg_ref[0, :, 0:1] + deg_ref[1, :, 0:1] - 1.0
    return lax.rsqrt(d)


def _tc_l1_body(x_ref, w_ref, deg_ref, o_ref):
    r = _dinv_block(deg_ref)
    o_ref[...] = r * jnp.dot(x_ref[...], w_ref[...],
                             preferred_element_type=jnp.float32)


def _tc_layer_body(acc_ref, u_ref, b_ref, w_ref, deg_ref, o_ref):
    r = _dinv_block(deg_ref)
    agg = acc_ref[0] + acc_ref[1] - u_ref[...]
    h = jnp.maximum(r * agg + b_ref[...], 0.0)
    o_ref[...] = r * jnp.dot(h, w_ref[...], preferred_element_type=jnp.float32)


def _tc_final_body(acc_ref, u_ref, b_ref, batch_ref, fw1_ref, fb1_ref,
                   fw2_ref, fb2_ref, deg_ref, o_ref, sums, cnt):
    i = pl.program_id(0)
    r = _dinv_block(deg_ref)
    h = jnp.maximum(r * (acc_ref[0] + acc_ref[1] - u_ref[...]) + b_ref[...], 0.0)
    onehot_t = (lax.broadcasted_iota(jnp.int32, (_G, 1), 0)
                == batch_ref[0]).astype(jnp.float32)

    @pl.when(i == 0)
    def _():
        sums[...] = jnp.zeros_like(sums)
        cnt[...] = jnp.zeros_like(cnt)

    sums[...] += jnp.dot(onehot_t, h, preferred_element_type=jnp.float32)
    cnt[...] += jnp.sum(onehot_t, axis=1, keepdims=True)

    @pl.when(i == pl.num_programs(0) - 1)
    def _():
        pooled = sums[...] / jnp.maximum(cnt[...], 1.0)
        hh = jnp.maximum(jnp.dot(pooled, fw1_ref[...],
                                 preferred_element_type=jnp.float32)
                         + fb1_ref[...], 0.0)
        o_ref[...] = jnp.dot(hh, fw2_ref[...],
                             preferred_element_type=jnp.float32) + fb2_ref[...]


def _row_spec(i):
    return (i, 0)


def _full_spec(i):
    return (0, 0)


def _acc_spec(i):
    return (0, i, 0)


_NGRID = _N // _RB


def _tc_l1(x, w1, degp):
    return pl.pallas_call(
        _tc_l1_body,
        grid=(_NGRID,),
        in_specs=[
            pl.BlockSpec((_RB, _D), _row_spec),
            pl.BlockSpec((_D, _D), _full_spec),
            pl.BlockSpec((_NC, _RB, _DW), _acc_spec),
        ],
        out_specs=pl.BlockSpec((_RB, _D), _row_spec),
        out_shape=jax.ShapeDtypeStruct((_NR, _D), jnp.float32),
    )(x, w1, degp)


def _tc_layer(acc, u, b, w, degp):
    return pl.pallas_call(
        _tc_layer_body,
        grid=(_NGRID,),
        in_specs=[
            pl.BlockSpec((_NC, _RB, _D), _acc_spec),
            pl.BlockSpec((_RB, _D), _row_spec),
            pl.BlockSpec((1, _D), _full_spec),
            pl.BlockSpec((_D, _D), _full_spec),
            pl.BlockSpec((_NC, _RB, _DW), _acc_spec),
        ],
        out_specs=pl.BlockSpec((_RB, _D), _row_spec),
        out_shape=jax.ShapeDtypeStruct((_NR, _D), jnp.float32),
    )(acc, u, b, w, degp)


def _tc_final(acc, u, b, batch2d, fw1, fb1, fw2p, fb2p, degp):
    return pl.pallas_call(
        _tc_final_body,
        grid=(_NGRID,),
        in_specs=[
            pl.BlockSpec((_NC, _RB, _D), _acc_spec),
            pl.BlockSpec((_RB, _D), _row_spec),
            pl.BlockSpec((1, _D), _full_spec),
            pl.BlockSpec((1, 1, _RB), lambda i: (i, 0, 0)),
            pl.BlockSpec((_D, _D), _full_spec),
            pl.BlockSpec((1, _D), _full_spec),
            pl.BlockSpec((_D, _D), _full_spec),
            pl.BlockSpec((1, _D), _full_spec),
            pl.BlockSpec((_NC, _RB, _DW), _acc_spec),
        ],
        out_specs=pl.BlockSpec((_G, _D), _full_spec),
        out_shape=jax.ShapeDtypeStruct((_G, _D), jnp.float32),
        scratch_shapes=[
            pltpu.VMEM((_G, _D), jnp.float32),
            pltpu.VMEM((_G, 1), jnp.float32),
        ],
    )(acc, u, b, batch2d, fw1, fb1, fw2p, fb2p, degp)


def kernel(x, edge_index, batch, W1, b1, W2, b2, W3, b3, fW1, fb1, fW2, fb2):
    src = edge_index[0]
    dst = edge_index[1]
    pad = _EP - _E
    src_p = jnp.concatenate([src, jnp.zeros((pad,), jnp.int32)])
    dst_p = jnp.concatenate([dst, jnp.full((pad,), _PAD_DST, jnp.int32)])
    ones_t = jnp.ones((_RPT, _DW), jnp.float32)

    sc_deg = _make_sc_deg()
    sc_agg = _make_sc_agg()

    degp = sc_deg(dst_p, ones_t)

    b1r = b1.reshape(1, _D)
    b2r = b2.reshape(1, _D)
    b3r = b3.reshape(1, _D)
    fb1r = fb1.reshape(1, _D)
    fw2p = jnp.zeros((_D, _D), jnp.float32).at[:, : fW2.shape[1]].set(fW2)
    fb2p = jnp.zeros((1, _D), jnp.float32).at[0, : fb2.shape[0]].set(fb2)
    batch2d = batch.reshape(_NGRID, 1, _RB)

    u1 = _tc_l1(x, W1, degp)
    acc1 = sc_agg(u1, src_p, dst_p)
    u2 = _tc_layer(acc1, u1, b1r, W2, degp)
    acc2 = sc_agg(u2, src_p, dst_p)
    u3 = _tc_layer(acc2, u2, b2r, W3, degp)
    acc3 = sc_agg(u3, src_p, dst_p)
    out = _tc_final(acc3, u3, b3r, batch2d, fW1, fb1r, fw2p, fb2p, degp)
    return out[:, : fW2.shape[1]]


# speedup vs baseline: 7.9343x; 1.2909x over previous
import functools

import jax
import jax.numpy as jnp
from jax import lax
from jax.experimental import pallas as pl
from jax.experimental.pallas import tpu as pltpu
from jax.experimental.pallas import tpu_sc as plsc

_N = 10000
_D = 128
_G = 16
_E = 320000
_NC = 2
_NS = 16
_CH = 128
_EPT = 10240
_EP = _NC * _NS * _EPT
_NCHUNK = _EPT // _CH
_NR = 10240
_PAD_DST = 10008
_RPT = _NR // _NS
_DW = 16
_RB = 1000


def _sc_mesh():
    return plsc.VectorSubcoreMesh(core_axis_name="c", subcore_axis_name="s")


def _sc_deg_body(dst_hbm, ones_hbm, deg_out, deg_sh, ones_v, idx_v):
    c = lax.axis_index("c")
    s = lax.axis_index("s")
    pltpu.sync_copy(ones_hbm.at[pl.ds(0, _CH)], ones_v)
    pltpu.sync_copy(ones_hbm, deg_sh.at[pl.ds(s * _RPT, _RPT)])
    plsc.subcore_barrier()
    tbase = c * (_EP // _NC) + s * _EPT

    def body(g, carry):
        base = tbase + g * _CH
        pltpu.sync_copy(dst_hbm.at[pl.ds(base, _CH)], idx_v.at[0])
        pltpu.sync_copy(ones_v, deg_sh.at[idx_v.at[0]], add=True)
        return carry

    lax.fori_loop(0, _NCHUNK, body, 0)
    plsc.subcore_barrier()
    pltpu.sync_copy(deg_sh.at[pl.ds(s * _RPT, _RPT)],
                    deg_out.at[c, pl.ds(s * _RPT, _RPT)])


def _make_sc_deg():
    return pl.kernel(
        _sc_deg_body,
        out_type=jax.ShapeDtypeStruct((_NC, _NR, _DW), jnp.float32),
        mesh=_sc_mesh(),
        scratch_types=[
            pltpu.VMEM_SHARED((_NR, _DW), jnp.float32),
            pltpu.VMEM((_CH, _DW), jnp.float32),
            pltpu.VMEM((1, _CH), jnp.int32),
        ],
        name="sc_deg",
    )


_ACH = 64
_ANCH = _EPT // _ACH
_NBUF = 5
_NIB = 10
_LOOK_I = 4
_LOOK_G = 2


def _sc_agg_body(u_hbm, sd_hbm, acc_out, acc_sh, idx_v, rows_v,
                 isem, gsem, ssem):
    c = lax.axis_index("c")
    s = lax.axis_index("s")
    pltpu.sync_copy(u_hbm.at[pl.ds(s * _RPT, _RPT)], acc_sh.at[pl.ds(s * _RPT, _RPT)])
    tchunk = (c * _NS + s) * _ANCH

    def fire_idx(ch, k):
        pltpu.async_copy(sd_hbm.at[tchunk + ch], idx_v.at[k], isem[k])

    def wait_idx(k):
        pltpu.make_async_copy(sd_hbm.at[tchunk], idx_v.at[k], isem[k]).wait()

    def fire_gather(k, b):
        pltpu.async_copy(u_hbm.at[idx_v.at[k, 0]], rows_v.at[b], gsem[b])

    def wait_gather(k, b):
        pltpu.make_async_copy(u_hbm.at[idx_v.at[k, 0]], rows_v.at[b],
                              gsem[b]).wait()

    def fire_scatter(k, b):
        pltpu.async_copy(rows_v.at[b], acc_sh.at[idx_v.at[k, 1]], ssem[b],
                         add=True)

    def wait_scatter(k, b):
        pltpu.make_async_copy(rows_v.at[b], acc_sh.at[idx_v.at[k, 1]],
                              ssem[b]).wait()

    for k in range(_LOOK_I):
        fire_idx(k, k)
    for k in range(_LOOK_G):
        wait_idx(k)
        fire_gather(k, k)
    plsc.subcore_barrier()

    def outer(go, carry):
        for j in range(_NIB):
            ch = go + j
            b = j % _NBUF
            wait_gather(j, b)
            fire_scatter(j, b)
            ki = (j + _LOOK_I) % _NIB
            kg = (j + _LOOK_G) % _NIB
            bg = (j + _LOOK_G) % _NBUF

            @pl.when(ch + _LOOK_I < _ANCH)
            def _():
                fire_idx(ch + _LOOK_I, ki)

            @pl.when(ch + _LOOK_G < _ANCH)
            def _():
                @pl.when(ch >= _NBUF - _LOOK_G)
                def _():
                    wait_scatter((j - (_NBUF - _LOOK_G)) % _NIB, bg)

                wait_idx(kg)
                fire_gather(kg, bg)

        return carry

    lax.fori_loop(0, _ANCH // _NIB, lambda i, cy: outer(i * _NIB, cy), 0)
    for j in range(_NIB - _NBUF, _NIB):
        wait_scatter(j, j % _NBUF)
    plsc.subcore_barrier()
    pltpu.sync_copy(acc_sh.at[pl.ds(s * _RPT, _RPT)],
                    acc_out.at[c, pl.ds(s * _RPT, _RPT)])


def _make_sc_agg():
    return pl.kernel(
        _sc_agg_body,
        out_type=jax.ShapeDtypeStruct((_NC, _NR, _D), jnp.float32),
        mesh=_sc_mesh(),
        scratch_types=[
            pltpu.VMEM_SHARED((_NR, _D), jnp.float32),
            pltpu.VMEM((_NIB, 2, _ACH), jnp.int32),
            pltpu.VMEM((_NBUF, _ACH, _D), jnp.float32),
            [pltpu.SemaphoreType.DMA] * _NIB,
            [pltpu.SemaphoreType.DMA] * _NBUF,
            [pltpu.SemaphoreType.DMA] * _NBUF,
        ],
        name="sc_agg",
    )


def _dinv_block(deg_ref):
    d = deg_ref[0, :, 0:1] + deg_ref[1, :, 0:1] - 1.0
    return lax.rsqrt(d)


def _tc_l1_body(x_ref, w_ref, deg_ref, o_ref):
    r = _dinv_block(deg_ref)
    o_ref[...] = r * jnp.dot(x_ref[...], w_ref[...],
                             preferred_element_type=jnp.float32)


def _tc_layer_body(acc_ref, u_ref, b_ref, w_ref, deg_ref, o_ref):
    r = _dinv_block(deg_ref)
    agg = acc_ref[0] + acc_ref[1] - u_ref[...]
    h = jnp.maximum(r * agg + b_ref[...], 0.0)
    o_ref[...] = r * jnp.dot(h, w_ref[...], preferred_element_type=jnp.float32)


def _tc_final_body(acc_ref, u_ref, b_ref, batch_ref, fw1_ref, fb1_ref,
                   fw2_ref, fb2_ref, deg_ref, o_ref, sums, cnt):
    i = pl.program_id(0)
    r = _dinv_block(deg_ref)
    h = jnp.maximum(r * (acc_ref[0] + acc_ref[1] - u_ref[...]) + b_ref[...], 0.0)
    onehot_t = (lax.broadcasted_iota(jnp.int32, (_G, 1), 0)
                == batch_ref[0]).astype(jnp.float32)

    @pl.when(i == 0)
    def _():
        sums[...] = jnp.zeros_like(sums)
        cnt[...] = jnp.zeros_like(cnt)

    sums[...] += jnp.dot(onehot_t, h, preferred_element_type=jnp.float32)
    cnt[...] += jnp.sum(onehot_t, axis=1, keepdims=True)

    @pl.when(i == pl.num_programs(0) - 1)
    def _():
        pooled = sums[...] / jnp.maximum(cnt[...], 1.0)
        hh = jnp.maximum(jnp.dot(pooled, fw1_ref[...],
                                 preferred_element_type=jnp.float32)
                         + fb1_ref[...], 0.0)
        o_ref[...] = jnp.dot(hh, fw2_ref[...],
                             preferred_element_type=jnp.float32) + fb2_ref[...]


def _row_spec(i):
    return (i, 0)


def _full_spec(i):
    return (0, 0)


def _acc_spec(i):
    return (0, i, 0)


_NGRID = _N // _RB


def _tc_l1(x, w1, degp):
    return pl.pallas_call(
        _tc_l1_body,
        grid=(_NGRID,),
        in_specs=[
            pl.BlockSpec((_RB, _D), _row_spec),
            pl.BlockSpec((_D, _D), _full_spec),
            pl.BlockSpec((_NC, _RB, _DW), _acc_spec),
        ],
        out_specs=pl.BlockSpec((_RB, _D), _row_spec),
        out_shape=jax.ShapeDtypeStruct((_NR, _D), jnp.float32),
    )(x, w1, degp)


def _tc_layer(acc, u, b, w, degp):
    return pl.pallas_call(
        _tc_layer_body,
        grid=(_NGRID,),
        in_specs=[
            pl.BlockSpec((_NC, _RB, _D), _acc_spec),
            pl.BlockSpec((_RB, _D), _row_spec),
            pl.BlockSpec((1, _D), _full_spec),
            pl.BlockSpec((_D, _D), _full_spec),
            pl.BlockSpec((_NC, _RB, _DW), _acc_spec),
        ],
        out_specs=pl.BlockSpec((_RB, _D), _row_spec),
        out_shape=jax.ShapeDtypeStruct((_NR, _D), jnp.float32),
    )(acc, u, b, w, degp)


def _tc_final(acc, u, b, batch2d, fw1, fb1, fw2p, fb2p, degp):
    return pl.pallas_call(
        _tc_final_body,
        grid=(_NGRID,),
        in_specs=[
            pl.BlockSpec((_NC, _RB, _D), _acc_spec),
            pl.BlockSpec((_RB, _D), _row_spec),
            pl.BlockSpec((1, _D), _full_spec),
            pl.BlockSpec((1, 1, _RB), lambda i: (i, 0, 0)),
            pl.BlockSpec((_D, _D), _full_spec),
            pl.BlockSpec((1, _D), _full_spec),
            pl.BlockSpec((_D, _D), _full_spec),
            pl.BlockSpec((1, _D), _full_spec),
            pl.BlockSpec((_NC, _RB, _DW), _acc_spec),
        ],
        out_specs=pl.BlockSpec((_G, _D), _full_spec),
        out_shape=jax.ShapeDtypeStruct((_G, _D), jnp.float32),
        scratch_shapes=[
            pltpu.VMEM((_G, _D), jnp.float32),
            pltpu.VMEM((_G, 1), jnp.float32),
        ],
    )(acc, u, b, batch2d, fw1, fb1, fw2p, fb2p, degp)


def kernel(x, edge_index, batch, W1, b1, W2, b2, W3, b3, fW1, fb1, fW2, fb2):
    src = edge_index[0]
    dst = edge_index[1]
    pad = _EP - _E
    src_p = jnp.concatenate([src, jnp.zeros((pad,), jnp.int32)])
    dst_p = jnp.concatenate([dst, jnp.full((pad,), _PAD_DST, jnp.int32)])
    sd = jnp.stack([src_p.reshape(_EP // _ACH, _ACH),
                    dst_p.reshape(_EP // _ACH, _ACH)], axis=1)
    ones_t = jnp.ones((_RPT, _DW), jnp.float32)

    sc_deg = _make_sc_deg()
    sc_agg = _make_sc_agg()

    degp = sc_deg(dst_p, ones_t)

    b1r = b1.reshape(1, _D)
    b2r = b2.reshape(1, _D)
    b3r = b3.reshape(1, _D)
    fb1r = fb1.reshape(1, _D)
    fw2p = jnp.zeros((_D, _D), jnp.float32).at[:, : fW2.shape[1]].set(fW2)
    fb2p = jnp.zeros((1, _D), jnp.float32).at[0, : fb2.shape[0]].set(fb2)
    batch2d = batch.reshape(_NGRID, 1, _RB)

    u1 = _tc_l1(x, W1, degp)
    acc1 = sc_agg(u1, sd)
    u2 = _tc_layer(acc1, u1, b1r, W2, degp)
    acc2 = sc_agg(u2, sd)
    u3 = _tc_layer(acc2, u2, b2r, W3, degp)
    acc3 = sc_agg(u3, sd)
    out = _tc_final(acc3, u3, b3r, batch2d, fW1, fb1r, fw2p, fb2p, degp)
    return out[:, : fW2.shape[1]]
